# Initial kernel scaffold; baseline (speedup 1.0000x reference)
import jax
import jax.numpy as jnp
from jax.experimental import pallas as pl


def kernel(x, batch, Wg, bg):
    raise NotImplementedError("write your pallas kernel here")


# speedup vs baseline: 6.1916x; 6.1916x over previous
import jax
import jax.numpy as jnp
from jax import lax
from jax.experimental import pallas as pl
from jax.experimental.pallas import tpu as pltpu
from jax.experimental.pallas import tpu_sc as plsc

N = 100000
D = 128
S = 512
NC = 2
NS = 16
NW = NC * NS
SEG_PER_W = S // NW
CH = 512
DV = D // 16

_NEG_INF = float("-inf")


def _seg_body(x_hbm, starts_hbm, wb_hbm, out_hbm, sbuf, wbuf, xbuf, gbuf, obuf):
    i32 = jnp.int32
    f32 = jnp.float32
    idx16 = lax.iota(i32, 16)
    wid = lax.axis_index("s") * NC + lax.axis_index("c")
    seg0 = wid * SEG_PER_W

    pltpu.sync_copy(starts_hbm.at[pl.ds(seg0, 32)], sbuf)
    pltpu.sync_copy(wb_hbm, wbuf)

    wv = [wbuf[pl.ds(16 * k, 16)] for k in range(DV)]
    b_s = jnp.sum(jnp.where(idx16 == 0, wbuf[pl.ds(D, 16)], 0.0))

    s_lo = sbuf[pl.ds(0, 16)]
    s_hi = sbuf[pl.ds(16, 16)]

    def isel(vec, k):
        return jnp.sum(jnp.where(idx16 == k, vec, 0))

    def gates_for_chunk(delta, v):
        def row(o, _):
            acc = xbuf[pl.ds(o * D, 16)] * wv[0]
            for k in range(1, DV):
                acc = acc + xbuf[pl.ds(o * D + 16 * k, 16)] * wv[k]
            g = jnp.sum(acc) + b_s
            plsc.store_scatter(gbuf, [jnp.broadcast_to(o, (16,))],
                               jnp.broadcast_to(g, (16,)), mask=idx16 == 0)
            return 0
        lax.fori_loop(delta, delta + v, row, 0)

    def chunk_geom(st, en, i):
        cs = st + i * CH
        dstart = jnp.minimum(cs, N - CH)
        delta = cs - dstart
        v = jnp.minimum(CH, en - cs)
        return dstart, delta, v

    def seg_body(k, _):
        st = isel(s_lo, k)
        en = isel(s_lo, k + 1) + isel(s_hi, k - 15)
        n = en - st
        nc = (n + CH - 1) // CH

        def p1(i, carry):
            m_run, s_run = carry
            dstart, delta, v = chunk_geom(st, en, i)
            pltpu.sync_copy(x_hbm.at[pl.ds(dstart * D, CH * D)], xbuf)
            gates_for_chunk(delta, v)
            jlo = delta // 16
            jhi = (delta + v + 15) // 16

            def maxstep(j, mvec):
                lane = 16 * j + idx16
                gv = gbuf[pl.ds(16 * j, 16)]
                gvm = jnp.where((lane >= delta) & (lane < delta + v), gv,
                                _NEG_INF)
                return jnp.maximum(mvec, gvm)
            mvec = lax.fori_loop(jlo, jhi, maxstep,
                                 jnp.full((16,), _NEG_INF, f32))
            m_new = jnp.maximum(m_run, jnp.max(mvec))

            def sumstep(j, svec):
                lane = 16 * j + idx16
                gv = gbuf[pl.ds(16 * j, 16)]
                gvm = jnp.where((lane >= delta) & (lane < delta + v), gv,
                                _NEG_INF)
                return svec + jnp.exp(gvm - m_new)
            svec = lax.fori_loop(jlo, jhi, sumstep, jnp.zeros((16,), f32))
            resc = jnp.exp(jnp.broadcast_to(m_run - m_new, (16,)))
            resc_s = jnp.sum(jnp.where(idx16 == 0, resc, 0.0))
            return m_new, s_run * resc_s + jnp.sum(svec)

        m_fin, s_fin = lax.fori_loop(
            0, nc, p1,
            (jnp.asarray(_NEG_INF, f32), jnp.asarray(0.0, f32)))
        inv_s = 1.0 / (jnp.broadcast_to(s_fin, (16,)) + 1e-16)

        def p2(i, acc):
            dstart, delta, v = chunk_geom(st, en, i)

            @pl.when(nc > 1)
            def _():
                pltpu.sync_copy(x_hbm.at[pl.ds(dstart * D, CH * D)], xbuf)
                gates_for_chunk(delta, v)

            jlo = delta // 16
            jhi = (delta + v + 15) // 16

            def wstep(j, _):
                gv = gbuf[pl.ds(16 * j, 16)]
                gbuf[pl.ds(16 * j, 16)] = jnp.exp(gv - m_fin) * inv_s
                return 0
            lax.fori_loop(jlo, jhi, wstep, 0)

            def row(o, acc):
                wbc = plsc.load_gather(gbuf, [jnp.broadcast_to(o, (16,))])
                return tuple(acc[kk] + xbuf[pl.ds(o * D + 16 * kk, 16)] * wbc
                             for kk in range(DV))
            return lax.fori_loop(delta, delta + v, row, acc)

        acc0 = tuple(jnp.zeros((16,), f32) for _ in range(DV))
        acc = lax.fori_loop(0, nc, p2, acc0)
        for kk in range(DV):
            obuf[pl.ds(k * D + 16 * kk, 16)] = acc[kk]
        return 0

    lax.fori_loop(0, SEG_PER_W, seg_body, 0)
    pltpu.sync_copy(obuf, out_hbm.at[pl.ds(seg0 * D, SEG_PER_W * D)])


def kernel(x, batch, Wg, bg):
    batch = batch.astype(jnp.int32)
    starts = jnp.searchsorted(
        batch, jnp.arange(S + 1, dtype=jnp.int32)).astype(jnp.int32)
    starts = jnp.concatenate(
        [starts, jnp.full((31,), N, jnp.int32)])
    wb = jnp.concatenate(
        [Wg.reshape(D).astype(jnp.float32),
         bg.reshape(1).astype(jnp.float32),
         jnp.zeros((15,), jnp.float32)])
    xf = x.reshape(-1)

    mesh = plsc.VectorSubcoreMesh(core_axis_name="c", subcore_axis_name="s",
                                  num_cores=NC, num_subcores=NS)
    run = pl.kernel(
        _seg_body,
        out_type=jax.ShapeDtypeStruct((S * D,), jnp.float32),
        mesh=mesh,
        compiler_params=pltpu.CompilerParams(needs_layout_passes=False),
        scratch_types=[
            pltpu.VMEM((32,), jnp.int32),
            pltpu.VMEM((144,), jnp.float32),
            pltpu.VMEM((CH * D,), jnp.float32),
            pltpu.VMEM((CH,), jnp.float32),
            pltpu.VMEM((SEG_PER_W * D,), jnp.float32),
        ],
    )
    outf = run(xf, starts, wb)
    return outf.reshape(S, D)


# speedup vs baseline: 8.2586x; 1.3338x over previous
import jax
import jax.numpy as jnp
from jax import lax
from jax.experimental import pallas as pl
from jax.experimental.pallas import tpu as pltpu
from jax.experimental.pallas import tpu_sc as plsc

N = 100000
D = 128
S = 512
NC = 2
NS = 16
NW = NC * NS
SEG_PER_W = S // NW
CH = 512
DV = D // 16

_NEG_INF = float("-inf")


def _seg_body(x_hbm, starts_hbm, wb_hbm, out_hbm, sbuf, wbuf, xbuf, gbuf, obuf):
    i32 = jnp.int32
    f32 = jnp.float32
    idx16 = lax.iota(i32, 16)
    wid = lax.axis_index("s") * NC + lax.axis_index("c")
    seg0 = wid * SEG_PER_W

    pltpu.sync_copy(starts_hbm.at[pl.ds(seg0, 32)], sbuf)
    pltpu.sync_copy(wb_hbm, wbuf)

    wv = [wbuf[pl.ds(16 * k, 16)] for k in range(DV)]
    b_s = jnp.sum(jnp.where(idx16 == 0, wbuf[pl.ds(D, 16)], 0.0))

    s_lo = sbuf[pl.ds(0, 16)]
    s_hi = sbuf[pl.ds(16, 16)]

    def isel(vec, k):
        return jnp.sum(jnp.where(idx16 == k, vec, 0))

    def gates_for_chunk(delta, v):
        @plsc.parallel_loop(delta, delta + v, unroll=8)
        def _row(o):
            acc = xbuf[pl.ds(o * D, 16)] * wv[0]
            for k in range(1, DV):
                acc = acc + xbuf[pl.ds(o * D + 16 * k, 16)] * wv[k]
            g = jnp.sum(acc) + b_s
            plsc.store_scatter(gbuf, [jnp.broadcast_to(o, (16,))],
                               jnp.broadcast_to(g, (16,)), mask=idx16 == 0)

    def chunk_geom(st, en, i):
        cs = st + i * CH
        dstart = jnp.minimum(cs, N - CH)
        delta = cs - dstart
        v = jnp.minimum(CH, en - cs)
        return dstart, delta, v

    def seg_body(k, _):
        st = isel(s_lo, k)
        en = isel(s_lo, k + 1) + isel(s_hi, k - 15)
        n = en - st
        nc = (n + CH - 1) // CH

        def p1(i, carry):
            m_run, s_run = carry
            dstart, delta, v = chunk_geom(st, en, i)
            pltpu.sync_copy(x_hbm.at[pl.ds(dstart * D, CH * D)], xbuf)
            gates_for_chunk(delta, v)
            jlo = delta // 16
            jhi = (delta + v + 15) // 16

            @plsc.parallel_loop(jlo, jhi, unroll=4,
                                carry=jnp.full((16,), _NEG_INF, f32))
            def mvec(j, mv):
                lane = 16 * j + idx16
                gv = gbuf[pl.ds(16 * j, 16)]
                gvm = jnp.where((lane >= delta) & (lane < delta + v), gv,
                                _NEG_INF)
                return jnp.maximum(mv, gvm)
            m_new = jnp.maximum(m_run, jnp.max(mvec))

            @plsc.parallel_loop(jlo, jhi, unroll=4,
                                carry=jnp.zeros((16,), f32))
            def svec(j, sv):
                lane = 16 * j + idx16
                gv = gbuf[pl.ds(16 * j, 16)]
                gvm = jnp.where((lane >= delta) & (lane < delta + v), gv,
                                _NEG_INF)
                return sv + jnp.exp(gvm - m_new)
            resc = jnp.exp(jnp.broadcast_to(m_run - m_new, (16,)))
            resc_s = jnp.sum(jnp.where(idx16 == 0, resc, 0.0))
            return m_new, s_run * resc_s + jnp.sum(svec)

        m_fin, s_fin = lax.fori_loop(
            0, nc, p1,
            (jnp.asarray(_NEG_INF, f32), jnp.asarray(0.0, f32)))
        inv_s = 1.0 / (jnp.broadcast_to(s_fin, (16,)) + 1e-16)

        def p2(i, acc):
            dstart, delta, v = chunk_geom(st, en, i)

            @pl.when(nc > 1)
            def _():
                pltpu.sync_copy(x_hbm.at[pl.ds(dstart * D, CH * D)], xbuf)
                gates_for_chunk(delta, v)

            jlo = delta // 16
            jhi = (delta + v + 15) // 16

            @plsc.parallel_loop(jlo, jhi, unroll=4)
            def _wstep(j):
                gv = gbuf[pl.ds(16 * j, 16)]
                gbuf[pl.ds(16 * j, 16)] = jnp.exp(gv - m_fin) * inv_s

            @plsc.parallel_loop(delta, delta + v, unroll=8, carry=acc)
            def row(o, acc):
                wbc = plsc.load_gather(gbuf, [jnp.broadcast_to(o, (16,))])
                return tuple(acc[kk] + xbuf[pl.ds(o * D + 16 * kk, 16)] * wbc
                             for kk in range(DV))
            return row

        acc0 = tuple(jnp.zeros((16,), f32) for _ in range(DV))
        acc = lax.fori_loop(0, nc, p2, acc0)
        for kk in range(DV):
            obuf[pl.ds(k * D + 16 * kk, 16)] = acc[kk]
        return 0

    lax.fori_loop(0, SEG_PER_W, seg_body, 0)
    pltpu.sync_copy(obuf, out_hbm.at[pl.ds(seg0 * D, SEG_PER_W * D)])


def kernel(x, batch, Wg, bg):
    batch = batch.astype(jnp.int32)
    starts = jnp.searchsorted(
        batch, jnp.arange(S + 1, dtype=jnp.int32)).astype(jnp.int32)
    starts = jnp.concatenate(
        [starts, jnp.full((31,), N, jnp.int32)])
    wb = jnp.concatenate(
        [Wg.reshape(D).astype(jnp.float32),
         bg.reshape(1).astype(jnp.float32),
         jnp.zeros((15,), jnp.float32)])
    xf = x.reshape(-1)

    mesh = plsc.VectorSubcoreMesh(core_axis_name="c", subcore_axis_name="s",
                                  num_cores=NC, num_subcores=NS)
    run = pl.kernel(
        _seg_body,
        out_type=jax.ShapeDtypeStruct((S * D,), jnp.float32),
        mesh=mesh,
        compiler_params=pltpu.CompilerParams(needs_layout_passes=False),
        scratch_types=[
            pltpu.VMEM((32,), jnp.int32),
            pltpu.VMEM((144,), jnp.float32),
            pltpu.VMEM((CH * D,), jnp.float32),
            pltpu.VMEM((CH,), jnp.float32),
            pltpu.VMEM((SEG_PER_W * D,), jnp.float32),
        ],
    )
    outf = run(xf, starts, wb)
    return outf.reshape(S, D)
